# Initial kernel scaffold; baseline (speedup 1.0000x reference)
import jax
import jax.numpy as jnp
from jax.experimental import pallas as pl


def kernel(x, table, gamma, beta):
    raise NotImplementedError("write your pallas kernel here")


# speedup vs baseline: 2.4473x; 2.4473x over previous
import functools

import jax
import jax.numpy as jnp
from jax import lax
from jax.experimental import pallas as pl
from jax.experimental.pallas import tpu as pltpu
from jax.experimental.pallas import tpu_sc as plsc

B = 4096
L = 50
D = 128
NROWS = B * L
NW = 32
RPW = NROWS // NW
CH = 128
NCH = RPW // CH
NV = D // 16
EPS = 1e-5


def _rsqrt(v):
    i = lax.bitcast_convert_type(v, jnp.int32)
    i = jnp.full((16,), 0x5F3759DF, jnp.int32) - lax.shift_right_logical(i, 1)
    y = lax.bitcast_convert_type(i, jnp.float32)
    for _ in range(3):
        y = y * (1.5 - 0.5 * v * y * y)
    return y


_GDN = lax.GatherDimensionNumbers(
    offset_dims=(), collapsed_slice_dims=(0,), start_index_map=(0,))


def _allsum(v):
    lane = lax.iota(jnp.int32, 16)
    for d in (8, 4, 2, 1):
        p = (lane ^ d).reshape(16, 1)
        v = v + lax.gather(v, p, _GDN, (1,),
                           mode=lax.GatherScatterMode.PROMISE_IN_BOUNDS)
    return v


def _body(x_hbm, table_hbm, gamma_hbm, beta_hbm, out_hbm,
          idx_v, buf, gam_v, bet_v, gsem):
    wid = lax.axis_index("s") * 2 + lax.axis_index("c")
    base = wid * RPW

    pltpu.sync_copy(x_hbm.at[wid], idx_v)
    pltpu.sync_copy(gamma_hbm, gam_v)
    pltpu.sync_copy(beta_hbm, bet_v)

    gs = [gam_v[pl.ds(16 * j, 16)] for j in range(NV)]
    bs = [bet_v[pl.ds(16 * j, 16)] for j in range(NV)]

    def row_body(r, c):
        vs = [buf[r, pl.ds(16 * j, 16)] for j in range(NV)]
        s = vs[0]
        q = vs[0] * vs[0]
        for j in range(1, NV):
            s = s + vs[j]
            q = q + vs[j] * vs[j]
        mean = _allsum(s) * (1.0 / D)
        var = _allsum(q) * (1.0 / D) - mean * mean
        inv = _rsqrt(var + EPS)
        for j in range(NV):
            buf[r, pl.ds(16 * j, 16)] = (vs[j] - mean) * inv * gs[j] + bs[j]
        return c

    def chunk_body(g, c):
        pltpu.async_copy(table_hbm.at[idx_v.at[g]], buf, gsem).wait()
        lax.fori_loop(0, CH, row_body, 0, unroll=2)
        pltpu.sync_copy(buf, out_hbm.at[pl.ds(base + g * CH, CH)])
        return c

    lax.fori_loop(0, NCH, chunk_body, 0)


@jax.jit
def _run(x3, table, gamma, beta):
    mesh = plsc.VectorSubcoreMesh(core_axis_name="c", subcore_axis_name="s")
    f = functools.partial(
        pl.kernel,
        mesh=mesh,
        out_type=jax.ShapeDtypeStruct((NROWS, D), jnp.float32),
        scratch_types=[
            pltpu.VMEM((NCH, CH), jnp.int32),
            pltpu.VMEM((CH, D), jnp.float32),
            pltpu.VMEM((D,), jnp.float32),
            pltpu.VMEM((D,), jnp.float32),
            pltpu.SemaphoreType.DMA,
        ],
    )(_body)
    return f(x3, table, gamma, beta)


def kernel(x, table, gamma, beta):
    out = _run(x.reshape(NW, NCH, CH), table, gamma, beta)
    return out.reshape(B, L, D)


# speedup vs baseline: 2.8090x; 1.1478x over previous
import functools

import jax
import jax.numpy as jnp
from jax import lax
from jax.experimental import pallas as pl
from jax.experimental.pallas import tpu as pltpu
from jax.experimental.pallas import tpu_sc as plsc

B = 4096
L = 50
D = 128
NROWS = B * L
NW = 32
RPW = NROWS // NW
CH = 128
NCH = RPW // CH
NBUF = 5
NV = D // 16
EPS = 1e-5


def _rsqrt(v):
    i = lax.bitcast_convert_type(v, jnp.int32)
    i = jnp.full((16,), 0x5F3759DF, jnp.int32) - lax.shift_right_logical(i, 1)
    y = lax.bitcast_convert_type(i, jnp.float32)
    for _ in range(3):
        y = y * (1.5 - 0.5 * v * y * y)
    return y


_GDN = lax.GatherDimensionNumbers(
    offset_dims=(), collapsed_slice_dims=(0,), start_index_map=(0,))


def _allsum(v):
    lane = lax.iota(jnp.int32, 16)
    for d in (8, 4, 2, 1):
        p = (lane ^ d).reshape(16, 1)
        v = v + lax.gather(v, p, _GDN, (1,),
                           mode=lax.GatherScatterMode.PROMISE_IN_BOUNDS)
    return v


def _body(x_hbm, table_hbm, gamma_hbm, beta_hbm, out_hbm,
          idx_v, bufs, gam_v, bet_v, g0, g1, g2, g3, g4, wsem):
    gsems = [g0, g1, g2, g3, g4]
    wid = lax.axis_index("s") * 2 + lax.axis_index("c")
    base = wid * RPW

    pltpu.sync_copy(x_hbm.at[wid], idx_v)
    pltpu.sync_copy(gamma_hbm, gam_v)
    pltpu.sync_copy(beta_hbm, bet_v)

    gs = [gam_v[pl.ds(16 * j, 16)] for j in range(NV)]
    bs = [bet_v[pl.ds(16 * j, 16)] for j in range(NV)]

    def compute(b):
        def row_body(r, c):
            vs = [bufs[b, r, pl.ds(16 * j, 16)] for j in range(NV)]
            s = vs[0]
            q = vs[0] * vs[0]
            for j in range(1, NV):
                s = s + vs[j]
                q = q + vs[j] * vs[j]
            mean = _allsum(s) * (1.0 / D)
            var = _allsum(q) * (1.0 / D) - mean * mean
            inv = _rsqrt(var + EPS)
            for j in range(NV):
                bufs[b, r, pl.ds(16 * j, 16)] = \
                    (vs[j] - mean) * inv * gs[j] + bs[j]
            return c

        lax.fori_loop(0, CH, row_body, 0, unroll=2)

    def turn(t, c):
        gcs = [pltpu.make_async_copy(
                   table_hbm.at[idx_v.at[t * NBUF + b]], bufs.at[b], gsems[b])
               for b in range(NBUF)]
        for gc in gcs:
            gc.start()
        wcs = []
        for b in range(NBUF):
            gcs[b].wait()
            compute(b)
            wc = pltpu.make_async_copy(
                bufs.at[b],
                out_hbm.at[pl.ds(base + (t * NBUF + b) * CH, CH)], wsem)
            wc.start()
            wcs.append(wc)
        for wc in wcs:
            wc.wait()
        return c

    lax.fori_loop(0, NCH // NBUF, turn, 0)


@jax.jit
def _run(x3, table, gamma, beta):
    mesh = plsc.VectorSubcoreMesh(core_axis_name="c", subcore_axis_name="s")
    f = functools.partial(
        pl.kernel,
        mesh=mesh,
        out_type=jax.ShapeDtypeStruct((NROWS, D), jnp.float32),
        scratch_types=[
            pltpu.VMEM((NCH, CH), jnp.int32),
            pltpu.VMEM((NBUF, CH, D), jnp.float32),
            pltpu.VMEM((D,), jnp.float32),
            pltpu.VMEM((D,), jnp.float32),
            pltpu.SemaphoreType.DMA,
            pltpu.SemaphoreType.DMA,
            pltpu.SemaphoreType.DMA,
            pltpu.SemaphoreType.DMA,
            pltpu.SemaphoreType.DMA,
            pltpu.SemaphoreType.DMA,
        ],
    )(_body)
    return f(x3, table, gamma, beta)


def kernel(x, table, gamma, beta):
    out = _run(x.reshape(NW, NCH, CH), table, gamma, beta)
    return out.reshape(B, L, D)


# speedup vs baseline: 2.9273x; 1.0421x over previous
import functools

import jax
import jax.numpy as jnp
from jax import lax
from jax.experimental import pallas as pl
from jax.experimental.pallas import tpu as pltpu
from jax.experimental.pallas import tpu_sc as plsc

B = 4096
L = 50
D = 128
NROWS = B * L
NW = 32
RPW = NROWS // NW
CH = 128
NCH = RPW // CH
NBUF = 5
NV = D // 16
EPS = 1e-5


def _rsqrt(v):
    i = lax.bitcast_convert_type(v, jnp.int32)
    i = jnp.full((16,), 0x5F3759DF, jnp.int32) - lax.shift_right_logical(i, 1)
    y = lax.bitcast_convert_type(i, jnp.float32)
    for _ in range(2):
        y = y * (1.5 - 0.5 * v * y * y)
    return y


_GDN = lax.GatherDimensionNumbers(
    offset_dims=(), collapsed_slice_dims=(0,), start_index_map=(0,))


def _allsum(v):
    lane = lax.iota(jnp.int32, 16)
    for d in (8, 4, 2, 1):
        p = (lane ^ d).reshape(16, 1)
        v = v + lax.gather(v, p, _GDN, (1,),
                           mode=lax.GatherScatterMode.PROMISE_IN_BOUNDS)
    return v


def _body(x_hbm, table_hbm, gamma_hbm, beta_hbm, out_hbm,
          idx_v, bufs, gam_v, bet_v, g0, g1, g2, g3, g4, wsem):
    gsems = [g0, g1, g2, g3, g4]
    wid = lax.axis_index("s") * 2 + lax.axis_index("c")
    base = wid * RPW

    pltpu.sync_copy(x_hbm.at[wid], idx_v)
    pltpu.sync_copy(gamma_hbm, gam_v)
    pltpu.sync_copy(beta_hbm, bet_v)

    gs = [gam_v[pl.ds(16 * j, 16)] for j in range(NV)]
    bs = [bet_v[pl.ds(16 * j, 16)] for j in range(NV)]

    def compute(b):
        def row_body(r, c):
            vs = [bufs[b, r, pl.ds(16 * j, 16)] for j in range(NV)]
            s = vs[0]
            q = vs[0] * vs[0]
            for j in range(1, NV):
                s = s + vs[j]
                q = q + vs[j] * vs[j]
            mean = _allsum(s) * (1.0 / D)
            var = _allsum(q) * (1.0 / D) - mean * mean
            inv = _rsqrt(var + EPS)
            for j in range(NV):
                bufs[b, r, pl.ds(16 * j, 16)] = \
                    (vs[j] - mean) * inv * gs[j] + bs[j]
            return c

        lax.fori_loop(0, CH, row_body, 0, unroll=2)

    def turn(t, c):
        gcs = [pltpu.make_async_copy(
                   table_hbm.at[idx_v.at[t * NBUF + b]], bufs.at[b], gsems[b])
               for b in range(NBUF)]
        for gc in gcs:
            gc.start()
        wcs = []
        for b in range(NBUF):
            gcs[b].wait()
            compute(b)
            wc = pltpu.make_async_copy(
                bufs.at[b],
                out_hbm.at[pl.ds(base + (t * NBUF + b) * CH, CH)], wsem)
            wc.start()
            wcs.append(wc)
        for wc in wcs:
            wc.wait()
        return c

    lax.fori_loop(0, NCH // NBUF, turn, 0)


@jax.jit
def _run(x3, table, gamma, beta):
    mesh = plsc.VectorSubcoreMesh(core_axis_name="c", subcore_axis_name="s")
    f = functools.partial(
        pl.kernel,
        mesh=mesh,
        out_type=jax.ShapeDtypeStruct((NROWS, D), jnp.float32),
        scratch_types=[
            pltpu.VMEM((NCH, CH), jnp.int32),
            pltpu.VMEM((NBUF, CH, D), jnp.float32),
            pltpu.VMEM((D,), jnp.float32),
            pltpu.VMEM((D,), jnp.float32),
            pltpu.SemaphoreType.DMA,
            pltpu.SemaphoreType.DMA,
            pltpu.SemaphoreType.DMA,
            pltpu.SemaphoreType.DMA,
            pltpu.SemaphoreType.DMA,
            pltpu.SemaphoreType.DMA,
        ],
    )(_body)
    return f(x3, table, gamma, beta)


def kernel(x, table, gamma, beta):
    out = _run(x.reshape(NW, NCH, CH), table, gamma, beta)
    return out.reshape(B, L, D)
